# Initial kernel scaffold; baseline (speedup 1.0000x reference)
import jax
import jax.numpy as jnp
from jax.experimental import pallas as pl


def kernel(center_words, target_words, negative_words, embedding_u, embedding_v):
    raise NotImplementedError("write your pallas kernel here")


# speedup vs baseline: 4.0326x; 4.0326x over previous
import functools

import jax
import jax.numpy as jnp
from jax import lax
from jax.experimental import pallas as pl
from jax.experimental.pallas import tpu as pltpu
from jax.experimental.pallas import tpu_sc as plsc

EMB = 64
LANES = 16
NC, NS = 2, 16
NW = NC * NS
CB = 64


def _sc_scores_kernel(B, K):
    KP1 = K + 1
    BPW = B // NW
    NCHUNK = BPW // CB
    mesh = plsc.VectorSubcoreMesh(core_axis_name="c", subcore_axis_name="s")

    @functools.partial(
        pl.kernel,
        out_type=jax.ShapeDtypeStruct((NW, NCHUNK, KP1 * CB), jnp.float32),
        mesh=mesh,
        scratch_types=[
            pltpu.VMEM((KP1, BPW), jnp.int32),
            pltpu.VMEM((BPW,), jnp.int32),
            pltpu.VMEM((CB, EMB), jnp.float32),
            pltpu.VMEM((KP1 * CB, EMB), jnp.float32),
            pltpu.VMEM((KP1 * CB,), jnp.float32),
            pltpu.SemaphoreType.DMA,
        ],
        compiler_params=pltpu.CompilerParams(
            needs_layout_passes=False, use_tc_tiling_on_sc=False),
    )
    def sc_kernel(uidx_hbm, cidx_hbm, emb_u, emb_v, out_hbm,
                  uidx_v, cidx_v, crows_v, urows_v, scores_v, sem):
        w = lax.axis_index("s") * NC + lax.axis_index("c")
        pltpu.sync_copy(uidx_hbm.at[w], uidx_v)
        pltpu.sync_copy(cidx_hbm.at[w], cidx_v)

        lanes = lax.iota(jnp.int32, LANES)

        def chunk_body(ch, _):
            base = ch * CB
            copies = [pltpu.async_copy(
                emb_v.at[cidx_v.at[pl.ds(base, CB)]], crows_v, sem)]
            for k in range(KP1):
                copies.append(pltpu.async_copy(
                    emb_u.at[uidx_v.at[k, pl.ds(base, CB)]],
                    urows_v.at[pl.ds(k * CB, CB)], sem))
            for c in copies:
                c.wait()

            for g in range(CB // LANES):
                blrow = g * LANES + lanes

                def d_body(d, accs):
                    dsp = jnp.full((LANES,), d, jnp.int32)
                    cd = plsc.load_gather(crows_v, [blrow, dsp])
                    return tuple(
                        accs[k] + plsc.load_gather(
                            urows_v, [blrow + (k * CB), dsp]) * cd
                        for k in range(KP1))

                accs = lax.fori_loop(
                    0, EMB, d_body,
                    tuple(jnp.zeros((LANES,), jnp.float32)
                          for _ in range(KP1)))
                for k in range(KP1):
                    scores_v[pl.ds(k * CB + g * LANES, LANES)] = (
                        accs[k] if k == 0 else -accs[k])

            pltpu.sync_copy(scores_v, out_hbm.at[w, ch])
            return 0

        lax.fori_loop(0, NCHUNK, chunk_body, 0)

    return sc_kernel


def _tc_loss_body(s_ref, o_ref):
    x = s_ref[...]
    ls = jnp.minimum(x, 0.0) - jnp.log(1.0 + jnp.exp(-jnp.abs(x)))
    o_ref[0, 0] = jnp.sum(ls)


def kernel(center_words, target_words, negative_words, embedding_u, embedding_v):
    B, K = negative_words.shape
    KP1 = K + 1
    BPW = B // NW

    u_idx = jnp.concatenate([target_words, negative_words], axis=1)
    u_idx = u_idx.reshape(NW, BPW, KP1).transpose(0, 2, 1)
    c_idx = center_words.reshape(NW, BPW)

    scores = _sc_scores_kernel(B, K)(u_idx, c_idx, embedding_u, embedding_v)
    total = B * KP1
    scores2d = scores.reshape(total // 128, 128)

    loss_sum = pl.pallas_call(
        _tc_loss_body,
        out_shape=jax.ShapeDtypeStruct((1, 1), jnp.float32),
        in_specs=[pl.BlockSpec(memory_space=pltpu.VMEM)],
        out_specs=pl.BlockSpec(memory_space=pltpu.SMEM),
    )(scores2d)
    return -loss_sum[0, 0] / B
